# baseline (speedup 1.0000x reference)
import functools

import jax
import jax.numpy as jnp
from jax import lax
from jax.experimental import pallas as pl
from jax.experimental.pallas import tpu as pltpu
from jax.experimental.pallas import tpu_sc as plsc

_NC = 2
_NS = 16
_NW = _NC * _NS
_K = 128
_BLK = 10240


def _cdiv(a, b):
    return (a + b - 1) // b


def _sc_mesh():
    return plsc.VectorSubcoreMesh(core_axis_name="c", subcore_axis_name="s")


def _deg_call(dst3, n_pad):
    nb = dst3.shape[1]
    rows_pt = n_pad // _NS

    @functools.partial(
        pl.kernel,
        out_type=jax.ShapeDtypeStruct((_NC, n_pad), jnp.float32),
        mesh=_sc_mesh(),
        scratch_types=[
            pltpu.VMEM((nb, _K), jnp.int32),
            pltpu.VMEM((_K,), jnp.float32),
            pltpu.VMEM((rows_pt,), jnp.float32),
            pltpu.VMEM_SHARED((n_pad,), jnp.float32),
            pltpu.SemaphoreType.DMA,
        ],
    )
    def k(dst_hbm, out_hbm, didx, ones, stage, acc, ssem):
        c = lax.axis_index("c")
        s = lax.axis_index("s")
        w = c * _NS + s
        pltpu.sync_copy(dst_hbm.at[w], didx)
        one16 = jnp.ones((16,), jnp.float32)
        zero16 = jnp.zeros((16,), jnp.float32)
        for j in range(_K // 16):
            ones[pl.ds(j * 16, 16)] = one16

        def zb(t, carry):
            stage[pl.ds(t * 16, 16)] = zero16
            return carry

        lax.fori_loop(0, rows_pt // 16, zb, 0)
        pltpu.sync_copy(stage, acc.at[pl.ds(s * rows_pt, rows_pt)])
        plsc.subcore_barrier()

        def body(i, carry):
            descs = [pltpu.async_copy(ones, acc.at[didx.at[8 * i + j]],
                                      ssem, add=True) for j in range(8)]
            for dsc in descs:
                dsc.wait()
            return carry

        lax.fori_loop(0, nb // 8, body, 0)
        plsc.subcore_barrier()
        pltpu.sync_copy(acc.at[pl.ds(s * rows_pt, rows_pt)],
                        out_hbm.at[c, pl.ds(s * rows_pt, rows_pt)])

    return k(dst3)


_CB = 8


def _prop_call(table, src3, dst3, w_dim):
    n_pad = table.shape[0]
    kb = 80
    nb = src3.shape[1] * src3.shape[2] // kb
    nch = nb // _CB
    src3 = src3.reshape(_NW, nb, kb)
    dst3 = dst3.reshape(_NW, nb, kb)
    rows_pt = n_pad // _NS
    zr = 16
    chunks = w_dim // 16

    @functools.partial(
        pl.kernel,
        out_type=jax.ShapeDtypeStruct((_NC, n_pad, w_dim), jnp.float32),
        mesh=_sc_mesh(),
        scratch_types=[
            pltpu.VMEM((2 * _CB, kb), jnp.int32),
            pltpu.VMEM((2 * _CB, kb), jnp.int32),
            pltpu.VMEM((kb, w_dim), jnp.float32),
            pltpu.VMEM((kb, w_dim), jnp.float32),
            pltpu.VMEM((kb, w_dim), jnp.float32),
            pltpu.VMEM((kb, w_dim), jnp.float32),
            pltpu.VMEM((zr, w_dim), jnp.float32),
            pltpu.VMEM_SHARED((n_pad, w_dim), jnp.float32),
        ] + [pltpu.SemaphoreType.DMA] * 11,
    )
    def k(tab_hbm, src_hbm, dst_hbm, out_hbm, sidx, didx, r0, r1, r2, r3,
          zbuf, acc, g0, g1, g2, g3, s0, s1, s2, s3, si_sem, di_sem, zsem):
        rows = [r0, r1, r2, r3]
        gs = [g0, g1, g2, g3]
        ss = [s0, s1, s2, s3]
        c = lax.axis_index("c")
        s = lax.axis_index("s")
        w = c * _NS + s
        pltpu.sync_copy(src_hbm.at[w, pl.ds(0, _CB)], sidx.at[pl.ds(0, _CB)])
        pltpu.sync_copy(dst_hbm.at[w, pl.ds(0, _CB)], didx.at[pl.ds(0, _CB)])
        pltpu.async_copy(tab_hbm.at[sidx.at[0]], rows[0], gs[0])
        pltpu.async_copy(tab_hbm.at[sidx.at[1]], rows[1], gs[1])
        zero16 = jnp.zeros((16,), jnp.float32)

        def zb(t, carry):
            zbuf[t // chunks, pl.ds((t % chunks) * 16, 16)] = zero16
            return carry

        lax.fori_loop(0, zr * chunks, zb, 0)
        zdescs = [pltpu.async_copy(zbuf, acc.at[pl.ds(s * rows_pt + r * zr, zr)],
                                   zsem) for r in range(rows_pt // zr)]
        for dsc in zdescs:
            dsc.wait()
        plsc.subcore_barrier()

        def wait_gather(x):
            pltpu.make_async_copy(tab_hbm.at[pl.ds(0, kb)], rows[x], gs[x]).wait()

        def wait_scatter(x):
            pltpu.make_async_copy(rows[x], acc.at[didx.at[0]], ss[x]).wait()

        def chunk(i, p, q, first):
            cn = lax.rem(i + 1, nch)
            d_si = pltpu.async_copy(src_hbm.at[w, pl.ds(cn * _CB, _CB)],
                                    sidx.at[pl.ds(q * _CB, _CB)], si_sem)
            d_di = pltpu.async_copy(dst_hbm.at[w, pl.ds(cn * _CB, _CB)],
                                    didx.at[pl.ds(q * _CB, _CB)], di_sem)
            for j in range(_CB):
                x = j % 4
                y = (j + 2) % 4
                wait_gather(x)
                pltpu.async_copy(rows[x], acc.at[didx.at[p * _CB + j]],
                                 ss[x], add=True)
                if not (first and j < 2):
                    wait_scatter(y)
                if j == 6:
                    d_si.wait()
                    d_di.wait()
                nxt = p * _CB + j + 2 if j < _CB - 2 else q * _CB + (j - 6)
                pltpu.async_copy(tab_hbm.at[sidx.at[nxt]], rows[y], gs[y])
            return 0

        chunk(0, 0, 1, True)

        def body(i, carry):
            pp = lax.rem(i, 2)
            return chunk(i, pp, 1 - pp, False)

        lax.fori_loop(1, nch, body, 0)
        wait_scatter((_CB - 2) % 4)
        wait_scatter((_CB - 1) % 4)
        wait_gather(0)
        wait_gather(1)
        plsc.subcore_barrier()
        pltpu.sync_copy(acc.at[pl.ds(s * rows_pt, rows_pt)],
                        out_hbm.at[c, pl.ds(s * rows_pt, rows_pt)])

    return k(table, src3, dst3)


def _scale_call(d0, d1, x, n_pad):
    d = x.shape[1]
    grid = (n_pad // _BLK,)

    def body(d0_ref, d1_ref, x_ref, dinv_ref, xs_ref):
        deg = d0_ref[...] + d1_ref[...] + 1.0
        dinv = lax.rsqrt(deg)
        dinv_ref[...] = dinv
        xs_ref[...] = x_ref[...] * dinv

    return pl.pallas_call(
        body,
        grid=grid,
        in_specs=[
            pl.BlockSpec((_BLK, 1), lambda i: (i, 0)),
            pl.BlockSpec((_BLK, 1), lambda i: (i, 0)),
            pl.BlockSpec((_BLK, d), lambda i: (i, 0)),
        ],
        out_specs=[
            pl.BlockSpec((_BLK, 1), lambda i: (i, 0)),
            pl.BlockSpec((_BLK, d), lambda i: (i, 0)),
        ],
        out_shape=[
            jax.ShapeDtypeStruct((n_pad, 1), jnp.float32),
            jax.ShapeDtypeStruct((n_pad, d), jnp.float32),
        ],
    )(d0, d1, x)


def _mid_call(s2, xs, dinv, w1a, b1a, w1b, b1b, w2a, w2b):
    n_pad, d = xs.shape
    h = w1a.shape[1]
    co = w2a.shape[1]
    grid = (n_pad // _BLK,)

    def body(s2_ref, xs_ref, dv_ref, w1a_ref, b1a_ref, w1b_ref,
             b1b_ref, w2a_ref, w2b_ref, ms_ref):
        dv = dv_ref[...]
        p = (s2_ref[0] + s2_ref[1] + xs_ref[...]) * dv
        ha = jnp.maximum(
            jnp.dot(p, w1a_ref[...], preferred_element_type=jnp.float32)
            + b1a_ref[...], 0.0)
        hb = jnp.maximum(
            jnp.dot(p, w1b_ref[...], preferred_element_type=jnp.float32)
            + b1b_ref[...], 0.0)
        ma = jnp.dot(ha, w2a_ref[...], preferred_element_type=jnp.float32)
        mb = jnp.dot(hb, w2b_ref[...], preferred_element_type=jnp.float32)
        zpad = jnp.zeros((ma.shape[0], d - 2 * co), jnp.float32)
        ms_ref[...] = jnp.concatenate([ma, mb, zpad], axis=1) * dv

    full = lambda shape: pl.BlockSpec(shape, lambda i: tuple(0 for _ in shape))
    return pl.pallas_call(
        body,
        grid=grid,
        in_specs=[
            pl.BlockSpec((2, _BLK, d), lambda i: (0, i, 0)),
            pl.BlockSpec((_BLK, d), lambda i: (i, 0)),
            pl.BlockSpec((_BLK, 1), lambda i: (i, 0)),
            full((d, h)), full((1, h)), full((d, h)), full((1, h)),
            full((h, co)), full((h, co)),
        ],
        out_specs=pl.BlockSpec((_BLK, d), lambda i: (i, 0)),
        out_shape=jax.ShapeDtypeStruct((n_pad, d), jnp.float32),
    )(s2, xs, dinv, w1a, b1a, w1b, b1b, w2a, w2b)


def _final_call(t2, ms, dinv, b2a, b2b, n):
    n_pad, d = ms.shape
    co = b2a.shape[1]
    grid = (n_pad // _BLK,)

    def body(t2_ref, ms_ref, dv_ref, ba_ref, bb_ref, qa_ref, qb_ref):
        q = (t2_ref[0] + t2_ref[1] + ms_ref[...]) * dv_ref[...]
        qa_ref[...] = q[:, :co] + ba_ref[...]
        qb_ref[...] = q[:, co:2 * co] + bb_ref[...]

    return pl.pallas_call(
        body,
        grid=grid,
        in_specs=[
            pl.BlockSpec((2, _BLK, d), lambda i: (0, i, 0)),
            pl.BlockSpec((_BLK, d), lambda i: (i, 0)),
            pl.BlockSpec((_BLK, 1), lambda i: (i, 0)),
            pl.BlockSpec((1, co), lambda i: (0, 0)),
            pl.BlockSpec((1, co), lambda i: (0, 0)),
        ],
        out_specs=[
            pl.BlockSpec((_BLK, co), lambda i: (i, 0)),
            pl.BlockSpec((_BLK, co), lambda i: (i, 0)),
        ],
        out_shape=[
            jax.ShapeDtypeStruct((n, co), jnp.float32),
            jax.ShapeDtypeStruct((n, co), jnp.float32),
        ],
    )(t2, ms, dinv, b2a, b2b)


def kernel(node_features, edge_indices, W1a, b1a, W2a, b2a, W1b, b1b, W2b, b2b):
    n, d = node_features.shape
    e = edge_indices.shape[1]
    c_out = W2a.shape[1]

    n_pad = _cdiv(n, _BLK) * _BLK
    nb = _cdiv(_cdiv(e, _NW * _K), 8) * 8
    e_pad = _NW * nb * _K

    padv = n + (jnp.arange(e_pad - e, dtype=jnp.int32) % (n_pad - n))
    src3 = jnp.concatenate([edge_indices[0], padv]).reshape(_NW, nb, _K)
    dst3 = jnp.concatenate([edge_indices[1], padv]).reshape(_NW, nb, _K)

    deg = _deg_call(dst3, n_pad)
    dinv, xs = _scale_call(deg[0][:, None], deg[1][:, None], node_features,
                           n_pad)
    s2 = _prop_call(xs, src3, dst3, d)
    ms = _mid_call(s2, xs, dinv, W1a, b1a.reshape(1, -1),
                   W1b, b1b.reshape(1, -1), W2a, W2b)
    t2 = _prop_call(ms, src3, dst3, d)
    return _final_call(t2, ms, dinv, b2a.reshape(1, -1), b2b.reshape(1, -1), n)


# speedup vs baseline: 1.0061x; 1.0061x over previous
import functools

import jax
import jax.numpy as jnp
from jax import lax
from jax.experimental import pallas as pl
from jax.experimental.pallas import tpu as pltpu
from jax.experimental.pallas import tpu_sc as plsc

_NC = 2
_NS = 16
_NW = _NC * _NS
_K = 128
_BLK = 2048


def _cdiv(a, b):
    return (a + b - 1) // b


def _sc_mesh():
    return plsc.VectorSubcoreMesh(core_axis_name="c", subcore_axis_name="s")


def _deg_call(dst3, n_pad):
    nb = dst3.shape[1]
    rows_pt = n_pad // _NS

    @functools.partial(
        pl.kernel,
        out_type=jax.ShapeDtypeStruct((_NC, n_pad), jnp.float32),
        mesh=_sc_mesh(),
        scratch_types=[
            pltpu.VMEM((nb, _K), jnp.int32),
            pltpu.VMEM((_K,), jnp.float32),
            pltpu.VMEM((rows_pt,), jnp.float32),
            pltpu.VMEM_SHARED((n_pad,), jnp.float32),
            pltpu.SemaphoreType.DMA,
        ],
    )
    def k(dst_hbm, out_hbm, didx, ones, stage, acc, ssem):
        c = lax.axis_index("c")
        s = lax.axis_index("s")
        w = c * _NS + s
        pltpu.sync_copy(dst_hbm.at[w], didx)
        one16 = jnp.ones((16,), jnp.float32)
        zero16 = jnp.zeros((16,), jnp.float32)
        for j in range(_K // 16):
            ones[pl.ds(j * 16, 16)] = one16

        def zb(t, carry):
            stage[pl.ds(t * 16, 16)] = zero16
            return carry

        lax.fori_loop(0, rows_pt // 16, zb, 0)
        pltpu.sync_copy(stage, acc.at[pl.ds(s * rows_pt, rows_pt)])
        plsc.subcore_barrier()

        def body(i, carry):
            descs = [pltpu.async_copy(ones, acc.at[didx.at[8 * i + j]],
                                      ssem, add=True) for j in range(8)]
            for dsc in descs:
                dsc.wait()
            return carry

        lax.fori_loop(0, nb // 8, body, 0)
        plsc.subcore_barrier()
        pltpu.sync_copy(acc.at[pl.ds(s * rows_pt, rows_pt)],
                        out_hbm.at[c, pl.ds(s * rows_pt, rows_pt)])

    return k(dst3)


_CB = 8


def _prop_call(table, src3, dst3, w_dim):
    n_pad = table.shape[0]
    kb = 80
    nb = src3.shape[1] * src3.shape[2] // kb
    nch = nb // _CB
    src3 = src3.reshape(_NW, nb, kb)
    dst3 = dst3.reshape(_NW, nb, kb)
    rows_pt = n_pad // _NS
    zr = 16
    chunks = w_dim // 16

    @functools.partial(
        pl.kernel,
        out_type=jax.ShapeDtypeStruct((_NC, n_pad, w_dim), jnp.float32),
        mesh=_sc_mesh(),
        scratch_types=[
            pltpu.VMEM((2 * _CB, kb), jnp.int32),
            pltpu.VMEM((2 * _CB, kb), jnp.int32),
            pltpu.VMEM((kb, w_dim), jnp.float32),
            pltpu.VMEM((kb, w_dim), jnp.float32),
            pltpu.VMEM((kb, w_dim), jnp.float32),
            pltpu.VMEM((kb, w_dim), jnp.float32),
            pltpu.VMEM((zr, w_dim), jnp.float32),
            pltpu.VMEM_SHARED((n_pad, w_dim), jnp.float32),
        ] + [pltpu.SemaphoreType.DMA] * 11,
    )
    def k(tab_hbm, src_hbm, dst_hbm, out_hbm, sidx, didx, r0, r1, r2, r3,
          zbuf, acc, g0, g1, g2, g3, s0, s1, s2, s3, si_sem, di_sem, zsem):
        rows = [r0, r1, r2, r3]
        gs = [g0, g1, g2, g3]
        ss = [s0, s1, s2, s3]
        c = lax.axis_index("c")
        s = lax.axis_index("s")
        w = c * _NS + s
        pltpu.sync_copy(src_hbm.at[w, pl.ds(0, _CB)], sidx.at[pl.ds(0, _CB)])
        pltpu.sync_copy(dst_hbm.at[w, pl.ds(0, _CB)], didx.at[pl.ds(0, _CB)])
        pltpu.async_copy(tab_hbm.at[sidx.at[0]], rows[0], gs[0])
        pltpu.async_copy(tab_hbm.at[sidx.at[1]], rows[1], gs[1])
        zero16 = jnp.zeros((16,), jnp.float32)

        def zb(t, carry):
            zbuf[t // chunks, pl.ds((t % chunks) * 16, 16)] = zero16
            return carry

        lax.fori_loop(0, zr * chunks, zb, 0)
        zdescs = [pltpu.async_copy(zbuf, acc.at[pl.ds(s * rows_pt + r * zr, zr)],
                                   zsem) for r in range(rows_pt // zr)]
        for dsc in zdescs:
            dsc.wait()
        plsc.subcore_barrier()

        def wait_gather(x):
            pltpu.make_async_copy(tab_hbm.at[pl.ds(0, kb)], rows[x], gs[x]).wait()

        def wait_scatter(x):
            pltpu.make_async_copy(rows[x], acc.at[didx.at[0]], ss[x]).wait()

        def chunk(i, p, q, first):
            cn = lax.rem(i + 1, nch)
            d_si = pltpu.async_copy(src_hbm.at[w, pl.ds(cn * _CB, _CB)],
                                    sidx.at[pl.ds(q * _CB, _CB)], si_sem)
            d_di = pltpu.async_copy(dst_hbm.at[w, pl.ds(cn * _CB, _CB)],
                                    didx.at[pl.ds(q * _CB, _CB)], di_sem)
            for j in range(_CB):
                x = j % 4
                y = (j + 2) % 4
                wait_gather(x)
                pltpu.async_copy(rows[x], acc.at[didx.at[p * _CB + j]],
                                 ss[x], add=True)
                if not (first and j < 2):
                    wait_scatter(y)
                if j == 6:
                    d_si.wait()
                    d_di.wait()
                nxt = p * _CB + j + 2 if j < _CB - 2 else q * _CB + (j - 6)
                pltpu.async_copy(tab_hbm.at[sidx.at[nxt]], rows[y], gs[y])
            return 0

        chunk(0, 0, 1, True)

        def body(i, carry):
            pp = lax.rem(i, 2)
            return chunk(i, pp, 1 - pp, False)

        lax.fori_loop(1, nch, body, 0)
        wait_scatter((_CB - 2) % 4)
        wait_scatter((_CB - 1) % 4)
        wait_gather(0)
        wait_gather(1)
        plsc.subcore_barrier()
        pltpu.sync_copy(acc.at[pl.ds(s * rows_pt, rows_pt)],
                        out_hbm.at[c, pl.ds(s * rows_pt, rows_pt)])

    return k(table, src3, dst3)


def _scale_call(d0, d1, x, n_pad):
    d = x.shape[1]
    grid = (n_pad // _BLK,)

    def body(d0_ref, d1_ref, x_ref, dinv_ref, xs_ref):
        deg = d0_ref[...] + d1_ref[...] + 1.0
        dinv = lax.rsqrt(deg)
        dinv_ref[...] = dinv
        xs_ref[...] = x_ref[...] * dinv

    return pl.pallas_call(
        body,
        grid=grid,
        in_specs=[
            pl.BlockSpec((_BLK, 1), lambda i: (i, 0)),
            pl.BlockSpec((_BLK, 1), lambda i: (i, 0)),
            pl.BlockSpec((_BLK, d), lambda i: (i, 0)),
        ],
        out_specs=[
            pl.BlockSpec((_BLK, 1), lambda i: (i, 0)),
            pl.BlockSpec((_BLK, d), lambda i: (i, 0)),
        ],
        out_shape=[
            jax.ShapeDtypeStruct((n_pad, 1), jnp.float32),
            jax.ShapeDtypeStruct((n_pad, d), jnp.float32),
        ],
    )(d0, d1, x)


def _mid_call(s2, xs, dinv, w1a, b1a, w1b, b1b, w2a, w2b):
    n_pad, d = xs.shape
    h = w1a.shape[1]
    co = w2a.shape[1]
    grid = (n_pad // _BLK,)

    def body(s2_ref, xs_ref, dv_ref, w1a_ref, b1a_ref, w1b_ref,
             b1b_ref, w2a_ref, w2b_ref, ms_ref):
        dv = dv_ref[...]
        p = (s2_ref[0] + s2_ref[1] + xs_ref[...]) * dv
        ha = jnp.maximum(
            jnp.dot(p, w1a_ref[...], preferred_element_type=jnp.float32)
            + b1a_ref[...], 0.0)
        hb = jnp.maximum(
            jnp.dot(p, w1b_ref[...], preferred_element_type=jnp.float32)
            + b1b_ref[...], 0.0)
        ma = jnp.dot(ha, w2a_ref[...], preferred_element_type=jnp.float32)
        mb = jnp.dot(hb, w2b_ref[...], preferred_element_type=jnp.float32)
        zpad = jnp.zeros((ma.shape[0], d - 2 * co), jnp.float32)
        ms_ref[...] = jnp.concatenate([ma, mb, zpad], axis=1) * dv

    full = lambda shape: pl.BlockSpec(shape, lambda i: tuple(0 for _ in shape))
    return pl.pallas_call(
        body,
        grid=grid,
        in_specs=[
            pl.BlockSpec((2, _BLK, d), lambda i: (0, i, 0)),
            pl.BlockSpec((_BLK, d), lambda i: (i, 0)),
            pl.BlockSpec((_BLK, 1), lambda i: (i, 0)),
            full((d, h)), full((1, h)), full((d, h)), full((1, h)),
            full((h, co)), full((h, co)),
        ],
        out_specs=pl.BlockSpec((_BLK, d), lambda i: (i, 0)),
        out_shape=jax.ShapeDtypeStruct((n_pad, d), jnp.float32),
    )(s2, xs, dinv, w1a, b1a, w1b, b1b, w2a, w2b)


def _final_call(t2, ms, dinv, b2a, b2b, n):
    n_pad, d = ms.shape
    co = b2a.shape[1]
    grid = (n_pad // _BLK,)

    def body(t2_ref, ms_ref, dv_ref, ba_ref, bb_ref, qa_ref, qb_ref):
        q = (t2_ref[0] + t2_ref[1] + ms_ref[...]) * dv_ref[...]
        qa_ref[...] = q[:, :co] + ba_ref[...]
        qb_ref[...] = q[:, co:2 * co] + bb_ref[...]

    return pl.pallas_call(
        body,
        grid=grid,
        in_specs=[
            pl.BlockSpec((2, _BLK, d), lambda i: (0, i, 0)),
            pl.BlockSpec((_BLK, d), lambda i: (i, 0)),
            pl.BlockSpec((_BLK, 1), lambda i: (i, 0)),
            pl.BlockSpec((1, co), lambda i: (0, 0)),
            pl.BlockSpec((1, co), lambda i: (0, 0)),
        ],
        out_specs=[
            pl.BlockSpec((_BLK, co), lambda i: (i, 0)),
            pl.BlockSpec((_BLK, co), lambda i: (i, 0)),
        ],
        out_shape=[
            jax.ShapeDtypeStruct((n, co), jnp.float32),
            jax.ShapeDtypeStruct((n, co), jnp.float32),
        ],
    )(t2, ms, dinv, b2a, b2b)


def kernel(node_features, edge_indices, W1a, b1a, W2a, b2a, W1b, b1b, W2b, b2b):
    n, d = node_features.shape
    e = edge_indices.shape[1]
    c_out = W2a.shape[1]

    n_pad = _cdiv(n, _BLK) * _BLK
    nb = _cdiv(_cdiv(e, _NW * _K), 8) * 8
    e_pad = _NW * nb * _K

    padv = n + (jnp.arange(e_pad - e, dtype=jnp.int32) % (n_pad - n))
    src3 = jnp.concatenate([edge_indices[0], padv]).reshape(_NW, nb, _K)
    dst3 = jnp.concatenate([edge_indices[1], padv]).reshape(_NW, nb, _K)

    deg = _deg_call(dst3, n_pad)
    dinv, xs = _scale_call(deg[0][:, None], deg[1][:, None], node_features,
                           n_pad)
    s2 = _prop_call(xs, src3, dst3, d)
    ms = _mid_call(s2, xs, dinv, W1a, b1a.reshape(1, -1),
                   W1b, b1b.reshape(1, -1), W2a, W2b)
    t2 = _prop_call(ms, src3, dst3, d)
    return _final_call(t2, ms, dinv, b2a.reshape(1, -1), b2b.reshape(1, -1), n)
